# Initial kernel scaffold; baseline (speedup 1.0000x reference)
import jax
import jax.numpy as jnp
from jax.experimental import pallas as pl


def kernel(x, edge_index, A0, B0, As, Bs, Wout, bout):
    raise NotImplementedError("write your pallas kernel here")


# speedup vs baseline: 5.9274x; 5.9274x over previous
import functools

import jax
import jax.numpy as jnp
from jax import lax
from jax.experimental import pallas as pl
from jax.experimental.pallas import tpu as pltpu
from jax.experimental.pallas import tpu_sc as plsc

NN = 10000
EE = 320000
HH = 32
DD = 128

NC = 2
NS = 16
NW = NC * NS
EPW = EE // NW
CH = 80
NCH = EPW // CH
ZR = 624
ZTAIL = NN - NS * ZR

_MESH = plsc.VectorSubcoreMesh(core_axis_name="c", subcore_axis_name="s")
_DIMS = (((1,), (1,)), ((), ()))


def _sc_agg_body(p_hbm, srcr, dstr, zeros_hbm, out_hbm,
                 acc, src_v, dst_v, rows, gsem):
    c = lax.axis_index("c")
    s = lax.axis_index("s")
    wid = c * NS + s

    pltpu.sync_copy(zeros_hbm.at[pl.ds(s * ZR, ZR)],
                    acc.at[pl.ds(s * ZR, ZR)])

    @pl.when(s == NS - 1)
    def _():
        pltpu.sync_copy(zeros_hbm.at[pl.ds(NS * ZR, ZTAIL)],
                        acc.at[pl.ds(NS * ZR, ZTAIL)])

    plsc.subcore_barrier()

    def body(j, _):
        pltpu.sync_copy(srcr.at[wid, j], src_v)
        pltpu.sync_copy(dstr.at[wid, j], dst_v)
        pltpu.async_copy(p_hbm.at[src_v], rows, gsem).wait()
        pltpu.sync_copy(rows, acc.at[dst_v], add=True)
        return ()

    lax.fori_loop(0, NCH, body, ())

    plsc.subcore_barrier()
    pltpu.sync_copy(acc.at[pl.ds(s * ZR, ZR)],
                    out_hbm.at[c, pl.ds(s * ZR, ZR)])

    @pl.when(s == NS - 1)
    def _():
        pltpu.sync_copy(acc.at[pl.ds(NS * ZR, ZTAIL)],
                        out_hbm.at[c, pl.ds(NS * ZR, ZTAIL)])


_sc_agg = pl.kernel(
    _sc_agg_body,
    out_type=jax.ShapeDtypeStruct((NC, NN, HH), jnp.float32),
    mesh=_MESH,
    compiler_params=pltpu.CompilerParams(use_tc_tiling_on_sc=False),
    scratch_types=[
        pltpu.VMEM_SHARED((NN, HH), jnp.float32),
        pltpu.VMEM((CH,), jnp.int32),
        pltpu.VMEM((CH,), jnp.int32),
        pltpu.VMEM((CH, HH), jnp.float32),
        pltpu.SemaphoreType.DMA,
    ],
)


def _sc_cnt_body(dstr, zeros_hbm, ones_hbm, out_hbm, acc, dst_v, ones_v):
    c = lax.axis_index("c")
    s = lax.axis_index("s")
    wid = c * NS + s

    pltpu.sync_copy(zeros_hbm.at[pl.ds(s * ZR, ZR)],
                    acc.at[pl.ds(s * ZR, ZR)])

    @pl.when(s == NS - 1)
    def _():
        pltpu.sync_copy(zeros_hbm.at[pl.ds(NS * ZR, ZTAIL)],
                        acc.at[pl.ds(NS * ZR, ZTAIL)])

    pltpu.sync_copy(ones_hbm, ones_v)
    plsc.subcore_barrier()

    def body(j, _):
        pltpu.sync_copy(dstr.at[wid, j], dst_v)
        pltpu.sync_copy(ones_v, acc.at[dst_v], add=True)
        return ()

    lax.fori_loop(0, NCH, body, ())

    plsc.subcore_barrier()
    pltpu.sync_copy(acc.at[pl.ds(s * ZR, ZR)],
                    out_hbm.at[c, pl.ds(s * ZR, ZR)])

    @pl.when(s == NS - 1)
    def _():
        pltpu.sync_copy(acc.at[pl.ds(NS * ZR, ZTAIL)],
                        out_hbm.at[c, pl.ds(NS * ZR, ZTAIL)])


_sc_cnt = pl.kernel(
    _sc_cnt_body,
    out_type=jax.ShapeDtypeStruct((NC, NN, 16), jnp.float32),
    mesh=_MESH,
    compiler_params=pltpu.CompilerParams(use_tc_tiling_on_sc=False),
    scratch_types=[
        pltpu.VMEM_SHARED((NN, 16), jnp.float32),
        pltpu.VMEM((CH,), jnp.int32),
        pltpu.VMEM((CH, 16), jnp.float32),
    ],
)


def _proj0_body(x_ref, a_ref, b_ref, cntp_ref, p_ref, q_ref, invc_ref):
    x = x_ref[...]
    p_ref[...] = lax.dot_general(x, a_ref[...], _DIMS,
                                 preferred_element_type=jnp.float32)
    q_ref[...] = lax.dot_general(x, b_ref[...], _DIMS,
                                 preferred_element_type=jnp.float32)
    cnt = cntp_ref[0, :, 0:1] + cntp_ref[1, :, 0:1]
    invc_ref[...] = jnp.broadcast_to(1.0 / jnp.maximum(cnt, 1.0), (NN, HH))


def _softmax1(v):
    m = jnp.max(v, axis=1, keepdims=True)
    e = jnp.exp(v - m)
    return e / jnp.sum(e, axis=1, keepdims=True)


def _softmax0(v):
    m = jnp.max(v, axis=0, keepdims=True)
    e = jnp.exp(v - m)
    return e / jnp.sum(e, axis=0, keepdims=True)


def _round_body(aggp_ref, q_ref, invc_ref, a_ref, b_ref, p_out, q_out, *,
                do_sm0):
    out = (aggp_ref[0] + aggp_ref[1]) * invc_ref[...] + q_ref[...]
    h = _softmax1(out)
    if do_sm0:
        h = _softmax0(h)
    p_out[...] = lax.dot_general(h, a_ref[...], _DIMS,
                                 preferred_element_type=jnp.float32)
    q_out[...] = lax.dot_general(h, b_ref[...], _DIMS,
                                 preferred_element_type=jnp.float32)


def _final_body(aggp_ref, q_ref, invc_ref, w_ref, bo_ref, y_ref):
    out = (aggp_ref[0] + aggp_ref[1]) * invc_ref[...] + q_ref[...]
    h = _softmax0(_softmax1(out))
    z = jnp.sum(h * w_ref[...], axis=1, keepdims=True) + bo_ref[0, 0]
    y_ref[...] = _softmax0(z)


_NH = jax.ShapeDtypeStruct((NN, HH), jnp.float32)

_proj0 = pl.pallas_call(
    _proj0_body, out_shape=(_NH, _NH, _NH))

_round_sm = pl.pallas_call(
    functools.partial(_round_body, do_sm0=True), out_shape=(_NH, _NH))
_round_nosm = pl.pallas_call(
    functools.partial(_round_body, do_sm0=False), out_shape=(_NH, _NH))

_final = pl.pallas_call(
    _final_body, out_shape=jax.ShapeDtypeStruct((NN, 1), jnp.float32))


def kernel(x, edge_index, A0, B0, As, Bs, Wout, bout):
    srcr = edge_index[0].reshape(NW, NCH, CH)
    dstr = edge_index[1].reshape(NW, NCH, CH)
    z32 = jnp.zeros((NN, HH), jnp.float32)
    z16 = jnp.zeros((NN, 16), jnp.float32)
    o16 = jnp.ones((CH, 16), jnp.float32)

    cntp = _sc_cnt(dstr, z16, o16)
    p, q, invc = _proj0(x, A0, B0, cntp)

    for r in range(16):
        aggp = _sc_agg(p, srcr, dstr, z32)
        rnd = _round_sm if r > 0 else _round_nosm
        p, q = rnd(aggp, q, invc, As[r], Bs[r])

    aggp = _sc_agg(p, srcr, dstr, z32)
    return _final(aggp, q, invc, Wout, bout.reshape(1, 1))


# speedup vs baseline: 20.6668x; 3.4866x over previous
import functools

import jax
import jax.numpy as jnp
from jax import lax
from jax.experimental import pallas as pl
from jax.experimental.pallas import tpu as pltpu
from jax.experimental.pallas import tpu_sc as plsc

NN = 10000
EE = 320000
HH = 32
DD = 128

NC = 2
NS = 16
NW = NC * NS
EPW = EE // NW
CH = 80
NCH = EPW // CH
ZR = 624
ZTAIL = NN - NS * ZR

_MESH = plsc.VectorSubcoreMesh(core_axis_name="c", subcore_axis_name="s")
_DIMS = (((1,), (1,)), ((), ()))


NB = 5
PG = 3


def _sc_agg_body(p_hbm, srcr, dstr, zeros_hbm, out_hbm,
                 acc, src_all, dst_all, rows, gsem, ssem):
    c = lax.axis_index("c")
    s = lax.axis_index("s")
    wid = c * NS + s

    def fire_gather(j, b):
        pltpu.async_copy(p_hbm.at[src_all.at[j]], rows.at[b], gsem.at[b])

    def wait_gather(j, b):
        pltpu.make_async_copy(p_hbm.at[src_all.at[j]], rows.at[b],
                              gsem.at[b]).wait()

    def fire_scatter(j, b):
        pltpu.async_copy(rows.at[b], acc.at[dst_all.at[j]], ssem.at[b],
                         add=True)

    def wait_scatter(j, b):
        pltpu.make_async_copy(rows.at[b], acc.at[dst_all.at[j]],
                              ssem.at[b]).wait()

    pltpu.sync_copy(srcr.at[wid], src_all)
    pltpu.sync_copy(dstr.at[wid], dst_all)
    pltpu.sync_copy(zeros_hbm.at[pl.ds(s * ZR, ZR)],
                    acc.at[pl.ds(s * ZR, ZR)])

    @pl.when(s == NS - 1)
    def _():
        pltpu.sync_copy(zeros_hbm.at[pl.ds(NS * ZR, ZTAIL)],
                        acc.at[pl.ds(NS * ZR, ZTAIL)])

    plsc.subcore_barrier()

    for j in range(PG):
        fire_gather(j, j % NB)

    for b in range(NB):
        jg = b + PG
        if b >= NB - PG:
            wait_scatter(b - (NB - PG), jg % NB)
        fire_gather(jg, jg % NB)
        wait_gather(b, b)
        fire_scatter(b, b)

    @pl.loop(NB, NCH - NB, step=NB)
    def _(j0):
        for b in range(NB):
            j = j0 + b
            jg = j + PG
            bg = (b + PG) % NB
            wait_scatter(jg - NB, bg)
            fire_gather(jg, bg)
            wait_gather(j, b)
            fire_scatter(j, b)

    for b in range(NB):
        j = NCH - NB + b
        jg = j + PG
        if jg < NCH:
            wait_scatter(jg - NB, jg % NB)
            fire_gather(jg, jg % NB)
        wait_gather(j, b)
        fire_scatter(j, b)
    for b in range(NB):
        wait_scatter(NCH - NB + b, b)

    plsc.subcore_barrier()
    pltpu.sync_copy(acc.at[pl.ds(s * ZR, ZR)],
                    out_hbm.at[c, pl.ds(s * ZR, ZR)])

    @pl.when(s == NS - 1)
    def _():
        pltpu.sync_copy(acc.at[pl.ds(NS * ZR, ZTAIL)],
                        out_hbm.at[c, pl.ds(NS * ZR, ZTAIL)])


_sc_agg = pl.kernel(
    _sc_agg_body,
    out_type=jax.ShapeDtypeStruct((NC, NN, HH), jnp.float32),
    mesh=_MESH,
    compiler_params=pltpu.CompilerParams(use_tc_tiling_on_sc=False),
    scratch_types=[
        pltpu.VMEM_SHARED((NN, HH), jnp.float32),
        pltpu.VMEM((NCH, CH), jnp.int32),
        pltpu.VMEM((NCH, CH), jnp.int32),
        pltpu.VMEM((NB, CH, HH), jnp.float32),
        pltpu.SemaphoreType.DMA((NB,)),
        pltpu.SemaphoreType.DMA((NB,)),
    ],
)


def _sc_cnt_body(dstr, zeros_hbm, ones_hbm, out_hbm, acc, dst_v, ones_v):
    c = lax.axis_index("c")
    s = lax.axis_index("s")
    wid = c * NS + s

    pltpu.sync_copy(zeros_hbm.at[pl.ds(s * ZR, ZR)],
                    acc.at[pl.ds(s * ZR, ZR)])

    @pl.when(s == NS - 1)
    def _():
        pltpu.sync_copy(zeros_hbm.at[pl.ds(NS * ZR, ZTAIL)],
                        acc.at[pl.ds(NS * ZR, ZTAIL)])

    pltpu.sync_copy(ones_hbm, ones_v)
    plsc.subcore_barrier()

    def body(j, _):
        pltpu.sync_copy(dstr.at[wid, j], dst_v)
        pltpu.sync_copy(ones_v, acc.at[dst_v], add=True)
        return ()

    lax.fori_loop(0, NCH, body, ())

    plsc.subcore_barrier()
    pltpu.sync_copy(acc.at[pl.ds(s * ZR, ZR)],
                    out_hbm.at[c, pl.ds(s * ZR, ZR)])

    @pl.when(s == NS - 1)
    def _():
        pltpu.sync_copy(acc.at[pl.ds(NS * ZR, ZTAIL)],
                        out_hbm.at[c, pl.ds(NS * ZR, ZTAIL)])


_sc_cnt = pl.kernel(
    _sc_cnt_body,
    out_type=jax.ShapeDtypeStruct((NC, NN, 16), jnp.float32),
    mesh=_MESH,
    compiler_params=pltpu.CompilerParams(use_tc_tiling_on_sc=False),
    scratch_types=[
        pltpu.VMEM_SHARED((NN, 16), jnp.float32),
        pltpu.VMEM((CH,), jnp.int32),
        pltpu.VMEM((CH, 16), jnp.float32),
    ],
)


def _proj0_body(x_ref, a_ref, b_ref, cntp_ref, p_ref, q_ref, invc_ref):
    x = x_ref[...]
    p_ref[...] = lax.dot_general(x, a_ref[...], _DIMS,
                                 preferred_element_type=jnp.float32)
    q_ref[...] = lax.dot_general(x, b_ref[...], _DIMS,
                                 preferred_element_type=jnp.float32)
    cnt = cntp_ref[0, :, 0:1] + cntp_ref[1, :, 0:1]
    invc_ref[...] = jnp.broadcast_to(1.0 / jnp.maximum(cnt, 1.0), (NN, HH))


def _softmax1(v):
    m = jnp.max(v, axis=1, keepdims=True)
    e = jnp.exp(v - m)
    return e / jnp.sum(e, axis=1, keepdims=True)


def _softmax0(v):
    m = jnp.max(v, axis=0, keepdims=True)
    e = jnp.exp(v - m)
    return e / jnp.sum(e, axis=0, keepdims=True)


def _round_body(aggp_ref, q_ref, invc_ref, a_ref, b_ref, p_out, q_out, *,
                do_sm0):
    out = (aggp_ref[0] + aggp_ref[1]) * invc_ref[...] + q_ref[...]
    h = _softmax1(out)
    if do_sm0:
        h = _softmax0(h)
    p_out[...] = lax.dot_general(h, a_ref[...], _DIMS,
                                 preferred_element_type=jnp.float32)
    q_out[...] = lax.dot_general(h, b_ref[...], _DIMS,
                                 preferred_element_type=jnp.float32)


def _final_body(aggp_ref, q_ref, invc_ref, w_ref, bo_ref, y_ref):
    out = (aggp_ref[0] + aggp_ref[1]) * invc_ref[...] + q_ref[...]
    h = _softmax0(_softmax1(out))
    z = jnp.sum(h * w_ref[...], axis=1, keepdims=True) + bo_ref[0, 0]
    y_ref[...] = _softmax0(z)


_NH = jax.ShapeDtypeStruct((NN, HH), jnp.float32)

_proj0 = pl.pallas_call(
    _proj0_body, out_shape=(_NH, _NH, _NH))

_round_sm = pl.pallas_call(
    functools.partial(_round_body, do_sm0=True), out_shape=(_NH, _NH))
_round_nosm = pl.pallas_call(
    functools.partial(_round_body, do_sm0=False), out_shape=(_NH, _NH))

_final = pl.pallas_call(
    _final_body, out_shape=jax.ShapeDtypeStruct((NN, 1), jnp.float32))


def kernel(x, edge_index, A0, B0, As, Bs, Wout, bout):
    srcr = edge_index[0].reshape(NW, NCH, CH)
    dstr = edge_index[1].reshape(NW, NCH, CH)
    z32 = jnp.zeros((NN, HH), jnp.float32)
    z16 = jnp.zeros((NN, 16), jnp.float32)
    o16 = jnp.ones((CH, 16), jnp.float32)

    cntp = _sc_cnt(dstr, z16, o16)
    p, q, invc = _proj0(x, A0, B0, cntp)

    for r in range(16):
        aggp = _sc_agg(p, srcr, dstr, z32)
        rnd = _round_sm if r > 0 else _round_nosm
        p, q = rnd(aggp, q, invc, As[r], Bs[r])

    aggp = _sc_agg(p, srcr, dstr, z32)
    return _final(aggp, q, invc, Wout, bout.reshape(1, 1))
